# baseline (speedup 1.0000x reference)
import functools

import jax
import jax.numpy as jnp
from jax import lax
from jax.experimental import pallas as pl
from jax.experimental.pallas import tpu as pltpu
from jax.experimental.pallas import tpu_sc as plsc

_NW = 32
_KB = 4


def _build_detile(V, D):
    C = V // 128
    TAIL = V - C * 128
    VP = (C + 1) * 128 if TAIL else C * 128
    T = (C + _NW - 1) // _NW
    mesh = plsc.VectorSubcoreMesh(core_axis_name="c", subcore_axis_name="s")

    @functools.partial(
        pl.kernel,
        mesh=mesh,
        out_type=jax.ShapeDtypeStruct((VP * D,), jnp.float32),
        compiler_params=pltpu.CompilerParams(
            use_tc_tiling_on_sc=True, needs_layout_passes=False),
        scratch_types=[
            pltpu.VMEM((D, 128), jnp.float32),
            pltpu.VMEM((D, 128), jnp.float32),
            pltpu.VMEM((128 * D,), jnp.float32),
            pltpu.VMEM((128 * D,), jnp.float32),
            pltpu.SemaphoreType.DMA,
            pltpu.SemaphoreType.DMA,
            pltpu.SemaphoreType.DMA,
            pltpu.SemaphoreType.DMA,
        ],
    )
    def body(et_hbm, tail_hbm, out_hbm, tin0, tin1, tout0, tout1,
             si0, si1, so0, so1):
        tins = (tin0, tin1)
        touts = (tout0, tout1)
        si = (si0, si1)
        so = (so0, so1)
        wid = lax.axis_index("s") * 2 + lax.axis_index("c")
        iota = lax.iota(jnp.int32, 16)
        iota_hi = iota + 16

        def chunk_of(t):
            return wid + _NW * t

        def fire_in(t, b):
            c = chunk_of(t)

            @pl.when(c < C)
            def _():
                pltpu.async_copy(
                    et_hbm.at[:, pl.ds(c * 128, 128)], tins[b], si[b])

        def tr_one(b, j, col0):
            col = jnp.full((16,), col0, jnp.int32)
            touts[b][pl.ds(D * j, 16)] = plsc.load_gather(
                tins[b], [iota, col])
            touts[b][pl.ds(D * j + 16, 16)] = plsc.load_gather(
                tins[b], [iota_hi, col])

        def process(t, b):
            c = chunk_of(t)
            fire_in(t + 1, 1 - b)

            @pl.when(c < C)
            def _():
                pltpu.make_async_copy(
                    et_hbm.at[:, pl.ds(0, 128)], tins[b], si[b]).wait()

            @pl.when((t >= 2) & (chunk_of(t - 2) < C))
            def _():
                pltpu.make_async_copy(
                    touts[b], out_hbm.at[pl.ds(0, 128 * D)], so[b]).wait()

            @pl.when(c < C)
            def _():
                def tr(j2, carry):
                    for u in range(4):
                        j = j2 * 4 + u
                        tr_one(b, j, j)
                    return carry

                lax.fori_loop(0, 32, tr, 0)
                pltpu.async_copy(
                    touts[b], out_hbm.at[pl.ds(c * 128 * D, 128 * D)],
                    so[b])

        fire_in(0, 0)

        def do_pair(i, carry):
            process(2 * i, 0)
            process(2 * i + 1, 1)
            return carry

        lax.fori_loop(0, (T + 3) // 2, do_pair, 0)
        if TAIL:
            j0 = 128 - TAIL

            @pl.when(wid == 0)
            def _():
                pltpu.sync_copy(tail_hbm, tins[0])

                def tr(j2, carry):
                    for u in range(4):
                        j = j2 * 4 + u
                        tr_one(0, j, j0 + j)
                    return carry

                lax.fori_loop(0, TAIL // 4, tr, 0)
                for j in range(TAIL - TAIL % 4, TAIL):
                    tr_one(0, j, j0 + j)
                pltpu.sync_copy(
                    touts[0], out_hbm.at[pl.ds(C * 128 * D, 128 * D)])

    return body


def _build_lookup(B, L, V, D):
    NB = B // 128
    ND = D // 8
    S = (L * NB) // (_NW * _KB)
    su_per_l = NB // _KB
    assert S % 2 == 0
    OBS = ND * _KB * 8 * 128

    mesh = plsc.VectorSubcoreMesh(core_axis_name="c", subcore_axis_name="s")

    @functools.partial(
        pl.kernel,
        mesh=mesh,
        out_type=jax.ShapeDtypeStruct((L * ND * NB * 8 * 128,), jnp.float32),
        compiler_params=pltpu.CompilerParams(
            use_tc_tiling_on_sc=False, needs_layout_passes=False),
        scratch_types=[
            pltpu.VMEM((2, _KB, 2, 128), jnp.int32),
            pltpu.VMEM((2, _KB * 128, D), jnp.float32),
            pltpu.VMEM((2, OBS), jnp.float32),
        ]
        + [pltpu.SemaphoreType.DMA] * (2 * _KB)
        + [pltpu.SemaphoreType.DMA] * 2
        + [pltpu.SemaphoreType.DMA] * 2,
    )
    def body(xl_hbm, e_hbm, ol_hbm, idxv, buf, obuf, *sems):
        sg = [sems[:_KB], sems[_KB:2 * _KB]]
        si = sems[2 * _KB:2 * _KB + 2]
        so = sems[2 * _KB + 2:]
        wid = lax.axis_index("s") * 2 + lax.axis_index("c")
        iota = lax.iota(jnp.int32, 16)
        dt_lo = lax.shift_right_logical(iota, 3)
        r_lo = lax.bitwise_and(iota, 7)
        base_lo = [dt_lo * (_KB * 1024) + k * 1024 + r_lo * 128
                   for k in range(_KB)]
        base_hi = [(dt_lo + 2) * (_KB * 1024) + k * 1024 + r_lo * 128
                   for k in range(_KB)]

        def unit_pos(su):
            return su // su_per_l, (su % su_per_l) * _KB

        def fire_g0(b, k):
            return pltpu.async_copy(
                e_hbm.at[idxv.at[b, k, 0]],
                buf.at[b, pl.ds(k * 128, 128)], sg[b][k])

        def fire_idx(b, su):
            l, bt0 = unit_pos(su)
            return pltpu.async_copy(
                xl_hbm.at[l, pl.ds(bt0, _KB)], idxv.at[b], si[b])

        def process(s, b):
            su = wid * S + s
            l, bt0 = unit_pos(su)
            @pl.when(s < S - 1)
            def _():
                pltpu.make_async_copy(
                    xl_hbm.at[0, pl.ds(0, _KB)], idxv.at[1 - b],
                    si[1 - b]).wait()
                for k in range(_KB):
                    fire_g0(1 - b, k)
            for k in range(_KB):
                pltpu.make_async_copy(
                    e_hbm.at[idxv.at[b, k, 0]],
                    buf.at[b, pl.ds(k * 128, 128)], sg[b][k]).wait()
                pltpu.async_copy(
                    e_hbm.at[idxv.at[b, k, 1]],
                    buf.at[b, pl.ds(k * 128, 128)], sg[b][k], add=True)
            @pl.when(s >= 2)
            def _():
                for dt in range(ND):
                    pltpu.make_async_copy(
                        obuf.at[b, pl.ds(0, _KB * 1024)],
                        ol_hbm.at[pl.ds(0, _KB * 1024)], so[b]).wait()
            for k in range(_KB):
                pltpu.make_async_copy(
                    e_hbm.at[idxv.at[b, k, 1]],
                    buf.at[b, pl.ds(k * 128, 128)], sg[b][k]).wait()

                def tr(j2, carry, _k=k):
                    i0, i1 = carry
                    for u in range(4):
                        j = j2 * 4 + u
                        plsc.store_scatter(
                            obuf.at[b], [i0],
                            buf[b, _k * 128 + j, pl.ds(0, 16)])
                        plsc.store_scatter(
                            obuf.at[b], [i1],
                            buf[b, _k * 128 + j, pl.ds(16, 16)])
                        i0 = i0 + 1
                        i1 = i1 + 1
                    return (i0, i1)

                lax.fori_loop(0, 32, tr, (base_lo[k], base_hi[k]))
            @pl.when(s < S - 2)
            def _():
                fire_idx(b, su + 2)
            for dt in range(ND):
                pltpu.async_copy(
                    obuf.at[b, pl.ds(dt * _KB * 1024, _KB * 1024)],
                    ol_hbm.at[pl.ds(((l * ND + dt) * NB + bt0) * 1024,
                                    _KB * 1024)], so[b])

        su0 = wid * S
        l0, b00 = unit_pos(su0)
        pltpu.sync_copy(xl_hbm.at[l0, pl.ds(b00, _KB)], idxv.at[0])
        for k in range(_KB):
            fire_g0(0, k)
        fire_idx(1, su0 + 1)

        def do_pair(i, carry):
            process(2 * i, 0)
            process(2 * i + 1, 1)
            return carry

        lax.fori_loop(0, S // 2, do_pair, 0)
        for b in range(2):
            for dt in range(ND):
                pltpu.make_async_copy(
                    obuf.at[b, pl.ds(0, _KB * 1024)],
                    ol_hbm.at[pl.ds(0, _KB * 1024)], so[b]).wait()

    return body


def kernel(x, E):
    B, L, H = x.shape
    V, D = E.shape
    assert H == 2 and D % 16 == 0 and B % 128 == 0
    NB, ND = B // 128, D // 8
    et = jnp.swapaxes(E, 0, 1)
    et_tail = lax.slice(et, (0, V - 128), (D, V))
    e_flat = _build_detile(V, D)(et, et_tail)
    e_lin = e_flat.reshape(e_flat.size // D, D)
    xl = (x.astype(jnp.int32)
          .transpose(1, 2, 0)
          .reshape(L, H, NB, 128)
          .transpose(0, 2, 1, 3))
    ol = _build_lookup(B, L, V, D)(xl, e_lin)
    return (ol.reshape(L, ND, NB, 8, 128)
            .transpose(2, 4, 0, 1, 3)
            .reshape(B, L, D))


# speedup vs baseline: 1.3844x; 1.3844x over previous
import functools

import jax
import jax.numpy as jnp
from jax import lax
from jax.experimental import pallas as pl
from jax.experimental.pallas import tpu as pltpu
from jax.experimental.pallas import tpu_sc as plsc

_NW = 32
_KB = 4


def _build_lookup(B, L, V, D):
    NB = B // 128
    S = (L * NB) // (_NW * _KB)
    su_per_l = NB // _KB
    P = _KB * 128
    assert S % 2 == 0

    mesh = plsc.VectorSubcoreMesh(core_axis_name="c", subcore_axis_name="s")

    @functools.partial(
        pl.kernel,
        mesh=mesh,
        out_type=jax.ShapeDtypeStruct((L, B, D), jnp.float32),
        compiler_params=pltpu.CompilerParams(
            use_tc_tiling_on_sc=False, needs_layout_passes=False),
        scratch_types=[
            pltpu.VMEM((2, _KB, 2, 128), jnp.int32),
            pltpu.VMEM((2, P, D), jnp.float32),
        ]
        + [pltpu.SemaphoreType.DMA] * (2 * _KB)
        + [pltpu.SemaphoreType.DMA] * 2
        + [pltpu.SemaphoreType.DMA] * 2,
    )
    def body(xl_hbm, e_hbm, ol_hbm, idxv, buf, *sems):
        sg = [sems[:_KB], sems[_KB:2 * _KB]]
        si = sems[2 * _KB:2 * _KB + 2]
        so = sems[2 * _KB + 2:]
        wid = lax.axis_index("s") * 2 + lax.axis_index("c")

        def unit_pos(su):
            return su // su_per_l, (su % su_per_l) * _KB

        def fire_g0(b, k):
            return pltpu.async_copy(
                e_hbm.at[idxv.at[b, k, 0]],
                buf.at[b, pl.ds(k * 128, 128)], sg[b][k])

        def fire_idx(b, su):
            l, bt0 = unit_pos(su)
            return pltpu.async_copy(
                xl_hbm.at[l, pl.ds(bt0, _KB)], idxv.at[b], si[b])

        def process(s, b):
            su = wid * S + s
            l, bt0 = unit_pos(su)
            @pl.when(s < S - 1)
            def _():
                pltpu.make_async_copy(
                    xl_hbm.at[0, pl.ds(0, _KB)], idxv.at[1 - b],
                    si[1 - b]).wait()
                for k in range(_KB):
                    fire_g0(1 - b, k)
            for k in range(_KB):
                pltpu.make_async_copy(
                    e_hbm.at[idxv.at[b, k, 0]],
                    buf.at[b, pl.ds(k * 128, 128)], sg[b][k]).wait()
                pltpu.async_copy(
                    e_hbm.at[idxv.at[b, k, 1]],
                    buf.at[b, pl.ds(k * 128, 128)], sg[b][k], add=True)
            @pl.when(s >= 2)
            def _():
                pltpu.make_async_copy(
                    buf.at[b], ol_hbm.at[0, pl.ds(0, P)], so[b]).wait()
            for k in range(_KB):
                pltpu.make_async_copy(
                    e_hbm.at[idxv.at[b, k, 1]],
                    buf.at[b, pl.ds(k * 128, 128)], sg[b][k]).wait()
            @pl.when(s < S - 2)
            def _():
                fire_idx(b, su + 2)
            pltpu.async_copy(
                buf.at[b], ol_hbm.at[l, pl.ds(bt0 * 128, P)], so[b])

        su0 = wid * S
        l0, b00 = unit_pos(su0)
        pltpu.sync_copy(xl_hbm.at[l0, pl.ds(b00, _KB)], idxv.at[0])
        for k in range(_KB):
            fire_g0(0, k)
        fire_idx(1, su0 + 1)

        def do_pair(i, carry):
            process(2 * i, 0)
            process(2 * i + 1, 1)
            return carry

        lax.fori_loop(0, S // 2, do_pair, 0)
        for b in range(2):
            pltpu.make_async_copy(
                buf.at[b], ol_hbm.at[0, pl.ds(0, P)], so[b]).wait()

    return body


def kernel(x, E):
    B, L, H = x.shape
    V, D = E.shape
    assert H == 2 and D % 16 == 0 and B % 128 == 0
    NB = B // 128
    xl = (x.astype(jnp.int32)
          .transpose(1, 2, 0)
          .reshape(L, H, NB, 128)
          .transpose(0, 2, 1, 3))
    ol = _build_lookup(B, L, V, D)(xl, E)
    return ol.transpose(1, 0, 2)
